# baseline (speedup 1.0000x reference)
import functools
import jax
import jax.numpy as jnp
from jax import lax
from jax.experimental import pallas as pl
from jax.experimental.pallas import tpu as pltpu
from jax.experimental.pallas import tpu_sc as plsc

NV = 32
CH = 8
L = 16
NW = 32
BW = 512
NCHUNK = BW // L


def _sc_body(x_hbm, o_hbm, idxv, obuf):
    wid = lax.axis_index("s") * 2 + lax.axis_index("c")
    base = wid * BW
    def _step(t, carry):
        pltpu.sync_copy(x_hbm.at[t, CH - 1, pl.ds(base, BW)], idxv)

        def _row(v, cc):
            for c in range(NCHUNK):
                iv = idxv[pl.ds(c * L, L)].astype(jnp.int32)
                obuf[v, pl.ds(c * L, L)] = jnp.where(iv == v, 1.0, 0.0)
            return cc

        lax.fori_loop(0, NV, _row, 0)
        pltpu.sync_copy(obuf, o_hbm.at[t, :, pl.ds(base, BW)])
        return carry

    lax.fori_loop(0, 200, _step, 0)


def kernel(network):
    B, T, C = network.shape
    xp = jnp.transpose(network, (1, 2, 0))
    mesh = plsc.VectorSubcoreMesh(core_axis_name="c", subcore_axis_name="s")
    k = functools.partial(
        pl.kernel,
        mesh=mesh,
        out_type=jax.ShapeDtypeStruct((T, NV, B), jnp.float32),
        compiler_params=pltpu.CompilerParams(use_tc_tiling_on_sc=False),
        scratch_types=[
            pltpu.VMEM((BW,), jnp.float32),
            pltpu.VMEM((NV, BW), jnp.float32),
        ],
    )(_sc_body)
    out = k(xp)
    return jnp.transpose(out, (2, 0, 1))


# speedup vs baseline: 16.7997x; 16.7997x over previous
import jax
import jax.numpy as jnp
from jax.experimental import pallas as pl
from jax.experimental.pallas import tpu as pltpu

NV = 32
CH = 8
TB = 10


def _in_copy(x_hbm, scr, sems, step, slot):
    return pltpu.make_async_copy(
        x_hbm.at[pl.ds(step * TB, TB), CH - 1:CH, :],
        scr.at[slot],
        sems.at[slot],
    )


def _onehot_body(x_hbm, o_ref, scr, sems):
    i = pl.program_id(0)
    slot = i % 2

    @pl.when(i == 0)
    def _():
        _in_copy(x_hbm, scr, sems, i, slot).start()

    @pl.when(i + 1 < pl.num_programs(0))
    def _():
        _in_copy(x_hbm, scr, sems, i + 1, 1 - slot).start()

    _in_copy(x_hbm, scr, sems, i, slot).wait()
    idx = scr[slot].astype(jnp.int32)
    v = jax.lax.broadcasted_iota(jnp.int32, o_ref.shape, 1)
    o_ref[...] = jnp.where(v == idx, 1.0, 0.0)


def kernel(network):
    B, T, C = network.shape
    xp = jnp.transpose(network, (1, 2, 0))
    out = pl.pallas_call(
        _onehot_body,
        grid=(T // TB,),
        in_specs=[pl.BlockSpec(memory_space=pl.MemorySpace.ANY)],
        out_specs=pl.BlockSpec((TB, NV, B), lambda i: (i, 0, 0)),
        out_shape=jax.ShapeDtypeStruct((T, NV, B), jnp.float32),
        scratch_shapes=[
            pltpu.VMEM((2, TB, 1, B), jnp.float32),
            pltpu.SemaphoreType.DMA((2,)),
        ],
    )(xp)
    return jnp.transpose(out, (2, 0, 1))
